# baseline (speedup 1.0000x reference)
import functools

import jax
import jax.numpy as jnp
from jax import lax
from jax.experimental import pallas as pl
from jax.experimental.pallas import tpu as pltpu
from jax.experimental.pallas import tpu_sc as plsc

D_MODEL = 64

_info = plsc.get_sparse_core_info()
_NC = _info.num_cores
_NS = _info.num_subcores
_NW = _NC * _NS


@functools.lru_cache(maxsize=None)
def _make_gather_t(S0, S1):
    NB = S0 // 128
    T_PER_W = NB // _NW
    JH = S1 // 2
    NBLK = JH * T_PER_W
    mesh = plsc.VectorSubcoreMesh(core_axis_name="c", subcore_axis_name="s")

    @functools.partial(
        pl.kernel,
        mesh=mesh,
        out_type=jax.ShapeDtypeStruct((S1, 8, NB, 8, 128), jnp.float32),
        scratch_types=[
            pltpu.VMEM((JH, 128 * T_PER_W), jnp.int32),
            pltpu.VMEM((128, D_MODEL), jnp.float32),
            pltpu.VMEM((128, D_MODEL), jnp.float32),
            pltpu.VMEM((8, 8, 128), jnp.float32),
            pltpu.VMEM((8, 8, 128), jnp.float32),
            pltpu.SemaphoreType.DMA,
            pltpu.SemaphoreType.DMA,
            pltpu.SemaphoreType.DMA,
            pltpu.SemaphoreType.DMA,
        ],
        compiler_params=pltpu.CompilerParams(
            use_tc_tiling_on_sc=False, needs_layout_passes=False),
    )
    def gather_kernel(xt_hbm, w_hbm, out_hbm, xslab, rows0, rows1,
                      tblk0, tblk1, g0, g1, o0, o1):
        rows = (rows0, rows1)
        tblk = (tblk0, tblk1)
        gs = (g0, g1)
        os_ = (o0, o1)
        wid = lax.axis_index("s") * _NC + lax.axis_index("c")
        tc0 = wid * T_PER_W

        lanes = lax.iota(jnp.int32, 16)
        idxc = [c0 * 16 + lanes for c0 in range(8)]

        def coords(i):
            jj = lax.shift_right_logical(i, 2)
            t = lax.bitwise_and(i, 3)
            return jj, t

        def gather_start(i, b):
            jj, t = coords(i)
            idx_ref = xslab.at[jj, pl.ds(t * 128, 128)]
            pltpu.async_copy(w_hbm.at[idx_ref], rows[b], gs[b])

        def gather_wait(b):
            pltpu.make_async_copy(
                w_hbm.at[xslab.at[0, pl.ds(0, 128)]], rows[b], gs[b]).wait()

        zero16 = jnp.zeros((16,), jnp.int32)

        def transpose(b):
            dvec = zero16
            for tr in range(8):
                for r in range(8):
                    vs = [plsc.load_gather(rows[b], [idxc[c0], dvec])
                          for c0 in range(8)]
                    for c0 in range(8):
                        tblk[b][tr, r, pl.ds(c0 * 16, 16)] = vs[c0]
                    dvec = dvec + 1

        def out_start(i, j0, b):
            jj, t = coords(i)
            pltpu.async_copy(tblk[b], out_hbm.at[j0 + jj, :, tc0 + t], os_[b])

        def out_wait(b):
            pltpu.make_async_copy(tblk[b], out_hbm.at[0, :, 0], os_[b]).wait()

        def half(h, carry):
            j0 = h * JH
            pltpu.sync_copy(
                xt_hbm.at[pl.ds(j0, JH), pl.ds(tc0 * 128, 128 * T_PER_W)],
                xslab)
            gather_start(0, 0)
            gather_start(1, 1)
            gather_wait(0)
            transpose(0)
            out_start(0, j0, 0)
            gather_start(2, 0)
            gather_wait(1)
            transpose(1)
            out_start(1, j0, 1)

            def body(k, carry2):
                i0 = 2 * k
                gather_start(i0 + 1, 1)
                gather_wait(0)
                out_wait(0)
                transpose(0)
                out_start(i0, j0, 0)

                @pl.when(k < NBLK // 2 - 1)
                def _():
                    gather_start(i0 + 2, 0)

                gather_wait(1)
                out_wait(1)
                transpose(1)
                out_start(i0 + 1, j0, 1)
                return carry2

            lax.fori_loop(1, NBLK // 2, body, 0)
            out_wait(0)
            out_wait(1)
            return carry

        lax.fori_loop(0, 2, half, 0)

    return gather_kernel


def kernel(x, W):
    S0, S1 = x.shape
    xt = jnp.swapaxes(x, 0, 1).astype(jnp.int32)
    out5 = _make_gather_t(S0, S1)(xt, W)
    return out5.transpose(2, 4, 0, 1, 3).reshape(S0, S1, D_MODEL)


# speedup vs baseline: 3.4445x; 3.4445x over previous
import functools

import jax
import jax.numpy as jnp
from jax import lax
from jax.experimental import pallas as pl
from jax.experimental.pallas import tpu as pltpu
from jax.experimental.pallas import tpu_sc as plsc

D_MODEL = 64

_info = plsc.get_sparse_core_info()
_NC = _info.num_cores
_NS = _info.num_subcores
_NW = _NC * _NS


@functools.lru_cache(maxsize=None)
def _make_gather_t(S0, S1):
    NB = S0 // 128
    T_PER_W = NB // _NW
    JH = S1 // 2
    NBLK = JH * T_PER_W
    mesh = plsc.VectorSubcoreMesh(core_axis_name="c", subcore_axis_name="s")

    @functools.partial(
        pl.kernel,
        mesh=mesh,
        out_type=jax.ShapeDtypeStruct((S1, 8, NB, 8, 128), jnp.float32),
        scratch_types=[
            pltpu.VMEM((JH, 128 * T_PER_W), jnp.int32),
            pltpu.VMEM((128, D_MODEL), jnp.float32),
            pltpu.VMEM((128, D_MODEL), jnp.float32),
            pltpu.VMEM((8, 8, 129), jnp.float32),
            pltpu.VMEM((8, 8, 129), jnp.float32),
            pltpu.SemaphoreType.DMA,
            pltpu.SemaphoreType.DMA,
            pltpu.SemaphoreType.DMA,
            pltpu.SemaphoreType.DMA,
        ],
        compiler_params=pltpu.CompilerParams(
            use_tc_tiling_on_sc=False, needs_layout_passes=False),
    )
    def gather_kernel(xt_hbm, w_hbm, out_hbm, xslab, rows0, rows1,
                      tblk0, tblk1, g0, g1, o0, o1):
        rows = (rows0, rows1)
        tblk = (tblk0, tblk1)
        gs = (g0, g1)
        os_ = (o0, o1)
        wid = lax.axis_index("s") * _NC + lax.axis_index("c")
        tc0 = wid * T_PER_W

        lanes = lax.iota(jnp.int32, 16)
        trv = [lax.shift_right_logical(d0 * 16 + lanes, 3) for d0 in range(4)]
        rv = [lax.bitwise_and(d0 * 16 + lanes, 7) for d0 in range(4)]

        def coords(i):
            jj = lax.shift_right_logical(i, 2)
            t = lax.bitwise_and(i, 3)
            return jj, t

        def gather_start(i, b):
            jj, t = coords(i)
            idx_ref = xslab.at[jj, pl.ds(t * 128, 128)]
            pltpu.async_copy(w_hbm.at[idx_ref], rows[b], gs[b])

        def gather_wait(b):
            pltpu.make_async_copy(
                w_hbm.at[xslab.at[0, pl.ds(0, 128)]], rows[b], gs[b]).wait()

        def transpose(b):
            def tbody(cq, carry):
                cbase = lax.broadcast(cq * 4, (16,))
                for cc in range(4):
                    c = 4 * cq + cc
                    cvec = cbase + cc
                    vs = [rows[b][c, pl.ds(d0 * 16, 16)] for d0 in range(4)]
                    for d0 in range(4):
                        plsc.store_scatter(tblk[b], [trv[d0], rv[d0], cvec],
                                           vs[d0])
                return carry

            lax.fori_loop(0, 32, tbody, 0)

        def out_start(i, j0, b):
            jj, t = coords(i)
            pltpu.async_copy(tblk[b].at[:, :, pl.ds(0, 128)],
                             out_hbm.at[j0 + jj, :, tc0 + t], os_[b])

        def out_wait(b):
            pltpu.make_async_copy(tblk[b].at[:, :, pl.ds(0, 128)],
                                  out_hbm.at[0, :, 0], os_[b]).wait()

        def half(h, carry):
            j0 = h * JH
            pltpu.sync_copy(
                xt_hbm.at[pl.ds(j0, JH), pl.ds(tc0 * 128, 128 * T_PER_W)],
                xslab)
            gather_start(0, 0)
            gather_start(1, 1)
            gather_wait(0)
            transpose(0)
            out_start(0, j0, 0)
            gather_start(2, 0)
            gather_wait(1)
            transpose(1)
            out_start(1, j0, 1)

            def body(k, carry2):
                i0 = 2 * k
                gather_start(i0 + 1, 1)
                gather_wait(0)
                out_wait(0)
                transpose(0)
                out_start(i0, j0, 0)

                @pl.when(k < NBLK // 2 - 1)
                def _():
                    gather_start(i0 + 2, 0)

                gather_wait(1)
                out_wait(1)
                transpose(1)
                out_start(i0 + 1, j0, 1)
                return carry2

            lax.fori_loop(1, NBLK // 2, body, 0)
            out_wait(0)
            out_wait(1)
            return carry

        lax.fori_loop(0, 2, half, 0)

    return gather_kernel


def kernel(x, W):
    S0, S1 = x.shape
    xt = jnp.swapaxes(x, 0, 1).astype(jnp.int32)
    out5 = _make_gather_t(S0, S1)(xt, W)
    return out5.transpose(2, 4, 0, 1, 3).reshape(S0, S1, D_MODEL)
